# baseline (speedup 1.0000x reference)
import functools

import jax
import jax.numpy as jnp
from jax import lax
from jax.experimental import pallas as pl
from jax.experimental.pallas import tpu as pltpu
from jax.experimental.pallas import tpu_sc as plsc

_NC = 2
_NS = 16
_NW = _NC * _NS
_LANES = 16
_NODES_PER_CHUNK = 4
_NBUF = 4


def _sc_neighbor_sums(tn, feat, S):
    nw, nch, rows_per_chunk = tn.shape
    _, D = feat.shape
    npc = rows_per_chunk // S
    cpw = nch * npc
    B = nw * cpw
    dv = D // _LANES
    mesh = plsc.VectorSubcoreMesh(
        core_axis_name="c", subcore_axis_name="s",
        num_cores=_NC, num_subcores=_NS)

    @functools.partial(
        pl.kernel,
        out_type=jax.ShapeDtypeStruct((B, D), jnp.float32),
        mesh=mesh,
        scratch_types=[
            pltpu.VMEM((nch, rows_per_chunk), jnp.int32),
            pltpu.VMEM((_NBUF, rows_per_chunk, D), jnp.float32),
            pltpu.VMEM((2, npc, D), jnp.float32),
            [pltpu.SemaphoreType.DMA] * _NBUF,
            [pltpu.SemaphoreType.DMA] * 2,
        ],
    )
    def sums_kernel(tn_hbm, feat_hbm, out_hbm, idx_v, rows_v, out_s,
                    sems, semo):
        wid = lax.axis_index("s") * _NC + lax.axis_index("c")
        pltpu.sync_copy(tn_hbm.at[wid], idx_v)
        for k in range(_NBUF):
            pltpu.async_copy(feat_hbm.at[idx_v.at[k]], rows_v.at[k], sems[k])

        def group(gc, carry):
            c0 = _NBUF * gc
            for k in range(_NBUF):
                c = c0 + k
                ko = k % 2
                pltpu.make_async_copy(
                    feat_hbm.at[idx_v.at[k]], rows_v.at[k], sems[k]).wait()

                @pl.when(c >= 2)
                def _():
                    pltpu.make_async_copy(
                        out_s.at[ko], out_hbm.at[pl.ds(0, npc)],
                        semo[ko]).wait()

                buf = rows_v.at[k]
                for j in range(npc):
                    def body(s, accs):
                        return tuple(
                            accs[d] + buf[j * S + s, pl.ds(d * _LANES, _LANES)]
                            for d in range(dv))
                    accs = lax.fori_loop(
                        0, S, body,
                        tuple(jnp.zeros((_LANES,), jnp.float32)
                              for _ in range(dv)),
                        unroll=8)
                    for d in range(dv):
                        out_s[ko, j, pl.ds(d * _LANES, _LANES)] = accs[d]
                pltpu.async_copy(
                    out_s.at[ko],
                    out_hbm.at[pl.ds(wid * cpw + c * npc, npc)], semo[ko])

                @pl.when(c + _NBUF < nch)
                def _():
                    pltpu.async_copy(
                        feat_hbm.at[idx_v.at[c + _NBUF]], rows_v.at[k],
                        sems[k])
            return carry

        lax.fori_loop(0, nch // _NBUF, group, 0)
        for ko in range(2):
            pltpu.make_async_copy(
                out_s.at[ko], out_hbm.at[pl.ds(0, npc)], semo[ko]).wait()

    return sums_kernel(tn, feat)


def _tc_linear(x, W, b, S):
    B, D_IN = x.shape
    D_OUT = W.shape[1]
    blk = min(B, 2048)
    scale = 1.0 / S

    def body(x_ref, w_ref, b_ref, o_ref):
        o_ref[...] = (
            jnp.dot(x_ref[...], w_ref[...], preferred_element_type=jnp.float32)
            * scale + b_ref[...])

    return pl.pallas_call(
        body,
        grid=(B // blk,),
        in_specs=[
            pl.BlockSpec((blk, D_IN), lambda i: (i, 0)),
            pl.BlockSpec((D_IN, D_OUT), lambda i: (0, 0)),
            pl.BlockSpec((1, D_OUT), lambda i: (0, 0)),
        ],
        out_specs=pl.BlockSpec((blk, D_OUT), lambda i: (i, 0)),
        out_shape=jax.ShapeDtypeStruct((B, D_OUT), jnp.float32),
    )(x, W, b.reshape(1, D_OUT))


def kernel(nodes, to_neighs, id2feat, W, b):
    B, S = to_neighs.shape
    half = B // 2
    rows_per_chunk = _NODES_PER_CHUNK * S
    nch = half // (_NW * _NODES_PER_CHUNK)
    tn = to_neighs.astype(jnp.int32)
    tn1 = tn[:half].reshape(_NW, nch, rows_per_chunk)
    tn2 = tn[half:].reshape(_NW, nch, rows_per_chunk)
    s1 = _sc_neighbor_sums(tn1, id2feat, S)
    s2 = _sc_neighbor_sums(tn2, id2feat, S)
    o1 = _tc_linear(s1, W, b, S)
    o2 = _tc_linear(s2, W, b, S)
    return jnp.concatenate([o1, o2], axis=0)


# speedup vs baseline: 1.0028x; 1.0028x over previous
import functools

import jax
import jax.numpy as jnp
from jax import lax
from jax.experimental import pallas as pl
from jax.experimental.pallas import tpu as pltpu
from jax.experimental.pallas import tpu_sc as plsc

_NC = 2
_NS = 16
_NW = _NC * _NS
_LANES = 16
_NBUF = 4


def _sc_neighbor_sums(tn, feat):
    B, S = tn.shape
    _, D = feat.shape
    cpw = B // _NW
    dv = D // _LANES
    ngrp = cpw // _NBUF
    mesh = plsc.VectorSubcoreMesh(
        core_axis_name="c", subcore_axis_name="s",
        num_cores=_NC, num_subcores=_NS)

    @functools.partial(
        pl.kernel,
        out_type=jax.ShapeDtypeStruct((B, D), jnp.float32),
        mesh=mesh,
        scratch_types=[
            pltpu.VMEM((cpw, S), jnp.int32),
            pltpu.VMEM((_NBUF, S, D), jnp.float32),
            pltpu.VMEM((2, _NBUF, D), jnp.float32),
            [pltpu.SemaphoreType.DMA] * _NBUF,
            [pltpu.SemaphoreType.DMA] * 2,
        ],
    )
    def sums_kernel(tn_hbm, feat_hbm, out_hbm, idx_v, rows_v, out_s,
                    sems, semo):
        wid = lax.axis_index("s") * _NC + lax.axis_index("c")
        pltpu.sync_copy(tn_hbm.at[pl.ds(wid * cpw, cpw)], idx_v)
        for k in range(_NBUF):
            pltpu.async_copy(feat_hbm.at[idx_v.at[k]], rows_v.at[k], sems[k])

        def pair(g2, carry):
            for half in range(2):
                g = 2 * g2 + half
                c0 = _NBUF * g

                @pl.when(g2 >= 1)
                def _():
                    pltpu.make_async_copy(
                        out_s.at[half], out_hbm.at[pl.ds(0, _NBUF)],
                        semo[half]).wait()

                for k in range(_NBUF):
                    c = c0 + k
                    pltpu.make_async_copy(
                        feat_hbm.at[idx_v.at[k]], rows_v.at[k],
                        sems[k]).wait()
                    buf = rows_v.at[k]

                    def body(s, accs):
                        return tuple(
                            accs[d] + buf[s, pl.ds(d * _LANES, _LANES)]
                            for d in range(dv))
                    accs = lax.fori_loop(
                        0, S, body,
                        tuple(jnp.zeros((_LANES,), jnp.float32)
                              for _ in range(dv)),
                        unroll=8)
                    for d in range(dv):
                        out_s[half, k, pl.ds(d * _LANES, _LANES)] = accs[d]

                    @pl.when(c + _NBUF < cpw)
                    def _():
                        pltpu.async_copy(
                            feat_hbm.at[idx_v.at[c + _NBUF]], rows_v.at[k],
                            sems[k])
                pltpu.async_copy(
                    out_s.at[half],
                    out_hbm.at[pl.ds(wid * cpw + c0, _NBUF)], semo[half])
            return carry

        lax.fori_loop(0, ngrp // 2, pair, 0)
        for half in range(2):
            pltpu.make_async_copy(
                out_s.at[half], out_hbm.at[pl.ds(0, _NBUF)],
                semo[half]).wait()

    return sums_kernel(tn, feat)


def _tc_linear(x, W, b, S):
    B, D_IN = x.shape
    D_OUT = W.shape[1]
    blk = min(B, 2048)
    scale = 1.0 / S

    def body(x_ref, w_ref, b_ref, o_ref):
        o_ref[...] = (
            jnp.dot(x_ref[...], w_ref[...], preferred_element_type=jnp.float32)
            * scale + b_ref[...])

    return pl.pallas_call(
        body,
        grid=(B // blk,),
        in_specs=[
            pl.BlockSpec((blk, D_IN), lambda i: (i, 0)),
            pl.BlockSpec((D_IN, D_OUT), lambda i: (0, 0)),
            pl.BlockSpec((1, D_OUT), lambda i: (0, 0)),
        ],
        out_specs=pl.BlockSpec((blk, D_OUT), lambda i: (i, 0)),
        out_shape=jax.ShapeDtypeStruct((B, D_OUT), jnp.float32),
    )(x, W, b.reshape(1, D_OUT))


def kernel(nodes, to_neighs, id2feat, W, b):
    _, S = to_neighs.shape
    sums = _sc_neighbor_sums(to_neighs.astype(jnp.int32), id2feat)
    return _tc_linear(sums, W, b, S)


# speedup vs baseline: 1.0507x; 1.0478x over previous
import functools

import jax
import jax.numpy as jnp
from jax import lax
from jax.experimental import pallas as pl
from jax.experimental.pallas import tpu as pltpu
from jax.experimental.pallas import tpu_sc as plsc

_NC = 2
_NS = 16
_NW = _NC * _NS
_LANES = 16
_NPC = 4
_NBUF = 3


def _sc_neighbor_sums(tn, feat):
    B, S = tn.shape
    _, D = feat.shape
    cpw = B // _NW
    dv = D // _LANES
    sv = S // _LANES
    nch = cpw // _NPC
    rpc = _NPC * S
    nfull = (nch // _NBUF) * _NBUF
    mesh = plsc.VectorSubcoreMesh(
        core_axis_name="c", subcore_axis_name="s",
        num_cores=_NC, num_subcores=_NS)
    iring = 8

    @functools.partial(
        pl.kernel,
        out_type=jax.ShapeDtypeStruct((B, D), jnp.float32),
        mesh=mesh,
        scratch_types=[
            pltpu.VMEM((cpw, S), jnp.int32),
            pltpu.VMEM((iring, rpc), jnp.int32),
            pltpu.VMEM((_NBUF, rpc, D), jnp.float32),
            pltpu.VMEM((_NBUF, _NPC, D), jnp.float32),
            [pltpu.SemaphoreType.DMA] * _NBUF,
            [pltpu.SemaphoreType.DMA] * _NBUF,
        ],
    )
    def sums_kernel(tn_hbm, feat_hbm, out_hbm, raw_v, idx_v, rows_v, out_s,
                    sems, semo):
        wid = lax.axis_index("s") * _NC + lax.axis_index("c")
        pltpu.sync_copy(tn_hbm.at[pl.ds(wid * cpw, cpw)], raw_v)

        def repack(c, slot):
            for a in range(_NPC):
                for h in range(sv):
                    idx_v[slot, pl.ds((a * sv + h) * _LANES, _LANES)] = (
                        raw_v[c * _NPC + a, pl.ds(h * _LANES, _LANES)])

        for k in range(_NBUF):
            repack(k, k)
            pltpu.async_copy(feat_hbm.at[idx_v.at[k]], rows_v.at[k], sems[k])

        def consume(c, k):
            pltpu.make_async_copy(
                feat_hbm.at[idx_v.at[k]], rows_v.at[k], sems[k]).wait()

            @pl.when(c >= _NBUF)
            def _():
                pltpu.make_async_copy(
                    out_s.at[k], out_hbm.at[pl.ds(0, _NPC)], semo[k]).wait()

            buf = rows_v.at[k]
            for j in range(_NPC):
                def body(s, accs):
                    return tuple(
                        accs[d] + buf[j * S + s, pl.ds(d * _LANES, _LANES)]
                        for d in range(dv))
                accs = lax.fori_loop(
                    0, S, body,
                    tuple(jnp.zeros((_LANES,), jnp.float32)
                          for _ in range(dv)),
                    unroll=8)
                for d in range(dv):
                    out_s[k, j, pl.ds(d * _LANES, _LANES)] = accs[d]
            pltpu.async_copy(
                out_s.at[k],
                out_hbm.at[pl.ds(wid * cpw + c * _NPC, _NPC)], semo[k])

            @pl.when(c + _NBUF < nch)
            def _():
                cn = c + _NBUF
                slot = lax.rem(cn, iring)
                repack(cn, slot)
                pltpu.async_copy(
                    feat_hbm.at[idx_v.at[slot]], rows_v.at[k], sems[k])

        def group(gc, carry):
            for k in range(_NBUF):
                consume(_NBUF * gc + k, k)
            return carry

        lax.fori_loop(0, nfull // _NBUF, group, 0)
        for k in range(nch - nfull):
            consume(nfull + k, k)
        for k in range(_NBUF):
            pltpu.make_async_copy(
                out_s.at[k], out_hbm.at[pl.ds(0, _NPC)], semo[k]).wait()

    return sums_kernel(tn, feat)


def _tc_linear(x, W, b, S):
    B, D_IN = x.shape
    D_OUT = W.shape[1]
    blk = min(B, 2048)
    scale = 1.0 / S

    def body(x_ref, w_ref, b_ref, o_ref):
        o_ref[...] = (
            jnp.dot(x_ref[...], w_ref[...], preferred_element_type=jnp.float32)
            * scale + b_ref[...])

    return pl.pallas_call(
        body,
        grid=(B // blk,),
        in_specs=[
            pl.BlockSpec((blk, D_IN), lambda i: (i, 0)),
            pl.BlockSpec((D_IN, D_OUT), lambda i: (0, 0)),
            pl.BlockSpec((1, D_OUT), lambda i: (0, 0)),
        ],
        out_specs=pl.BlockSpec((blk, D_OUT), lambda i: (i, 0)),
        out_shape=jax.ShapeDtypeStruct((B, D_OUT), jnp.float32),
    )(x, W, b.reshape(1, D_OUT))


def kernel(nodes, to_neighs, id2feat, W, b):
    _, S = to_neighs.shape
    sums = _sc_neighbor_sums(to_neighs.astype(jnp.int32), id2feat)
    return _tc_linear(sums, W, b, S)


# speedup vs baseline: 1.0713x; 1.0196x over previous
import functools

import jax
import jax.numpy as jnp
from jax import lax
from jax.experimental import pallas as pl
from jax.experimental.pallas import tpu as pltpu
from jax.experimental.pallas import tpu_sc as plsc

_NC = 2
_NS = 16
_NW = _NC * _NS
_LANES = 16
_NPC = 4
_NBUF = 3
_FLUSH = 8


def _sc_neighbor_sums(tn, feat):
    B, S = tn.shape
    _, D = feat.shape
    cpw = B // _NW
    dv = D // _LANES
    sv = S // _LANES
    nch = cpw // _NPC
    rpc = _NPC * S
    nfull = (nch // _NBUF) * _NBUF
    mesh = plsc.VectorSubcoreMesh(
        core_axis_name="c", subcore_axis_name="s",
        num_cores=_NC, num_subcores=_NS)
    iring = 8

    @functools.partial(
        pl.kernel,
        out_type=jax.ShapeDtypeStruct((B, D), jnp.float32),
        mesh=mesh,
        scratch_types=[
            pltpu.VMEM((cpw, S), jnp.int32),
            pltpu.VMEM((iring, rpc), jnp.int32),
            pltpu.VMEM((_NBUF, rpc, D), jnp.float32),
            pltpu.VMEM((2 * _FLUSH * _NPC, D), jnp.float32),
            [pltpu.SemaphoreType.DMA] * _NBUF,
            pltpu.SemaphoreType.DMA,
        ],
    )
    def sums_kernel(tn_hbm, feat_hbm, out_hbm, raw_v, idx_v, rows_v, out_b,
                    sems, semo):
        wid = lax.axis_index("s") * _NC + lax.axis_index("c")
        pltpu.sync_copy(tn_hbm.at[pl.ds(wid * cpw, cpw)], raw_v)

        def repack(c, slot):
            for a in range(_NPC):
                for h in range(sv):
                    idx_v[slot, pl.ds((a * sv + h) * _LANES, _LANES)] = (
                        raw_v[c * _NPC + a, pl.ds(h * _LANES, _LANES)])

        for k in range(_NBUF):
            repack(k, k)
            pltpu.async_copy(feat_hbm.at[idx_v.at[k]], rows_v.at[k], sems[k])

        frows = _FLUSH * _NPC

        def consume(c, k):
            pltpu.make_async_copy(
                feat_hbm.at[idx_v.at[k]], rows_v.at[k], sems[k]).wait()

            cb = lax.rem(c, 2 * _FLUSH)

            @pl.when(jnp.logical_and(lax.rem(c, _FLUSH) == 0,
                                     c >= 2 * _FLUSH))
            def _():
                pltpu.make_async_copy(
                    out_b.at[pl.ds(0, frows)],
                    out_hbm.at[pl.ds(0, frows)], semo).wait()

            buf = rows_v.at[k]
            for j in range(_NPC):
                def body(s, accs):
                    return tuple(
                        accs[d] + buf[j * S + s, pl.ds(d * _LANES, _LANES)]
                        for d in range(dv))
                accs = lax.fori_loop(
                    0, S, body,
                    tuple(jnp.zeros((_LANES,), jnp.float32)
                          for _ in range(dv)),
                    unroll=8)
                for d in range(dv):
                    out_b[cb * _NPC + j, pl.ds(d * _LANES, _LANES)] = accs[d]

            @pl.when(lax.rem(c, _FLUSH) == _FLUSH - 1)
            def _():
                start = pl.multiple_of((cb - (_FLUSH - 1)) * _NPC, frows)
                hstart = pl.multiple_of(
                    wid * cpw + (c - (_FLUSH - 1)) * _NPC, frows)
                pltpu.async_copy(
                    out_b.at[pl.ds(start, frows)],
                    out_hbm.at[pl.ds(hstart, frows)], semo)

            @pl.when(c + _NBUF < nch)
            def _():
                cn = c + _NBUF
                slot = lax.rem(cn, iring)
                repack(cn, slot)
                pltpu.async_copy(
                    feat_hbm.at[idx_v.at[slot]], rows_v.at[k], sems[k])

        def group(gc, carry):
            for k in range(_NBUF):
                consume(_NBUF * gc + k, k)
            return carry

        lax.fori_loop(0, nfull // _NBUF, group, 0)
        for k in range(nch - nfull):
            consume(nfull + k, k)
        for _ in range(2):
            pltpu.make_async_copy(
                out_b.at[pl.ds(0, frows)],
                out_hbm.at[pl.ds(0, frows)], semo).wait()

    return sums_kernel(tn, feat)


def _tc_linear(x, W, b, S):
    B, D_IN = x.shape
    D_OUT = W.shape[1]
    blk = min(B, 4096)
    scale = 1.0 / S

    def body(x_ref, w_ref, b_ref, o_ref):
        o_ref[...] = (
            jnp.dot(x_ref[...], w_ref[...], preferred_element_type=jnp.float32)
            * scale + b_ref[...])

    return pl.pallas_call(
        body,
        grid=(B // blk,),
        in_specs=[
            pl.BlockSpec((blk, D_IN), lambda i: (i, 0)),
            pl.BlockSpec((D_IN, D_OUT), lambda i: (0, 0)),
            pl.BlockSpec((1, D_OUT), lambda i: (0, 0)),
        ],
        out_specs=pl.BlockSpec((blk, D_OUT), lambda i: (i, 0)),
        out_shape=jax.ShapeDtypeStruct((B, D_OUT), jnp.float32),
    )(x, W, b.reshape(1, D_OUT))


def kernel(nodes, to_neighs, id2feat, W, b):
    _, S = to_neighs.shape
    sums = _sc_neighbor_sums(to_neighs.astype(jnp.int32), id2feat)
    return _tc_linear(sums, W, b, S)
